# baseline (speedup 1.0000x reference)
import jax
import jax.numpy as jnp
from jax.experimental import pallas as pl
from jax.experimental.pallas import tpu as pltpu
from jax.experimental.pallas import tpu_sc as plsc

D_MODEL = 64
SCALE = 8.0
WINDOW = 128
LANES = 16


def kernel(x, table):
    B, S = x.shape
    N = B * S
    idx = x.reshape(1, N).astype(jnp.int32)
    mesh = plsc.VectorSubcoreMesh(core_axis_name="c", subcore_axis_name="s")

    @pl.kernel(
        out_type=jax.ShapeDtypeStruct((N, D_MODEL), jnp.float32),
        mesh=mesh,
        compiler_params=pltpu.CompilerParams(use_tc_tiling_on_sc=False),
    )
    def emb_kernel(tbl_hbm, i_hbm, o_hbm):
        def body(i_vmem, o_vmem):
            pltpu.sync_copy(tbl_hbm.at[i_vmem.at[0]], o_vmem)

            @pl.loop(0, WINDOW)
            def _(r):
                @pl.loop(0, D_MODEL, step=LANES)
                def _(c):
                    slc = (pl.ds(r, 1), pl.ds(c, LANES))
                    o_vmem.at[*slc][...] = o_vmem.at[*slc][...] * SCALE

        pltpu.emit_pipeline(
            body,
            grid=(N // WINDOW,),
            in_specs=[pl.BlockSpec((1, WINDOW), lambda i: (0, i))],
            out_specs=[pl.BlockSpec((WINDOW, D_MODEL), lambda i: (i, 0))],
            core_axis_name=("c", "s"),
            dimension_semantics=(pltpu.PARALLEL,),
        )(i_hbm, o_hbm)

    out = emb_kernel(table, idx)
    return out.reshape(B, S, D_MODEL)
